# Initial kernel scaffold; baseline (speedup 1.0000x reference)
import jax
import jax.numpy as jnp
from jax.experimental import pallas as pl


def kernel(x, embedding):
    raise NotImplementedError("write your pallas kernel here")


# speedup vs baseline: 1.0459x; 1.0459x over previous
import functools

import jax
import jax.numpy as jnp
from jax import lax
from jax.experimental import pallas as pl
from jax.experimental.pallas import tpu as pltpu
from jax.experimental.pallas import tpu_sc as plsc

DIM = 32
K = 8192
N = 8192
TOKEN_BLOCK = 256


_CHUNK_K = 2048


def _dist_argmin_body(x_ref, embt_ref, codes_ref):
    xb = x_ref[...]
    embt = embt_ref[...]
    a_sq = jnp.sum(xb * xb, axis=-1, keepdims=True)
    b_sq = jnp.sum(embt * embt, axis=0, keepdims=True)
    prod = lax.dot_general(
        xb, embt, (((1,), (0,)), ((), ())),
        preferred_element_type=jnp.float32,
    )
    dist = a_sq + b_sq - 2 * prod
    best_v = None
    best_i = None
    for c in range(K // _CHUNK_K):
        dslice = dist[:, c * _CHUNK_K:(c + 1) * _CHUNK_K]
        cmin = jnp.min(dslice, axis=1, keepdims=True)
        iota = lax.broadcasted_iota(jnp.int32, dslice.shape, 1) + c * _CHUNK_K
        cidx = jnp.min(jnp.where(dslice == cmin, iota, K), axis=1, keepdims=True)
        if c == 0:
            best_v, best_i = cmin, cidx
        else:
            keep = (best_v < cmin) | ((best_v == cmin) & (best_i < cidx))
            best_v = jnp.where(keep, best_v, cmin)
            best_i = jnp.where(keep, best_i, cidx)
        best_v = best_v.astype(jnp.bfloat16).astype(jnp.float32)
    codes_ref[...] = best_i


def _dist_argmin(xf, embt):
    return pl.pallas_call(
        _dist_argmin_body,
        grid=(N // TOKEN_BLOCK,),
        in_specs=[
            pl.BlockSpec((TOKEN_BLOCK, DIM), lambda i: (i, 0)),
            pl.BlockSpec((DIM, K), lambda i: (0, 0)),
        ],
        out_specs=pl.BlockSpec((TOKEN_BLOCK, 1), lambda i: (i, 0)),
        out_shape=jax.ShapeDtypeStruct((N, 1), jnp.int32),
    )(xf, embt)


_NC, _NS = 2, 16
_NW = _NC * _NS
_BPW = N // _NW
_CHUNK = 128
_NCH = _BPW // _CHUNK

@functools.lru_cache(maxsize=1)
def _make_sc_gather():
    mesh = plsc.VectorSubcoreMesh(core_axis_name="c", subcore_axis_name="s")

    @functools.partial(
        pl.kernel,
        mesh=mesh,
        out_type=jax.ShapeDtypeStruct((N, DIM), jnp.float32),
        scratch_types=[
            pltpu.VMEM((_NCH, _CHUNK), jnp.int32),
            pltpu.VMEM((_NCH, _CHUNK, DIM), jnp.float32),
            pltpu.SemaphoreType.DMA,
        ],
        compiler_params=pltpu.CompilerParams(use_tc_tiling_on_sc=False),
    )
    def _sc_gather(table_hbm, idx_hbm, out_hbm, idx_v, rows_v, sem):
        wid = lax.axis_index("s") * _NC + lax.axis_index("c")
        pltpu.sync_copy(idx_hbm.at[pl.ds(wid * _NCH, _NCH)], idx_v)
        copies = [
            pltpu.async_copy(table_hbm.at[idx_v.at[j]], rows_v.at[j], sem)
            for j in range(_NCH)
        ]
        for c in copies:
            c.wait()
        for j in range(_NCH):
            pltpu.sync_copy(
                rows_v.at[j], out_hbm.at[pl.ds(wid * _BPW + j * _CHUNK, _CHUNK)]
            )

    return _sc_gather


def kernel(x, embedding):
    xf = jnp.reshape(x, (-1, x.shape[-1]))
    embt = embedding.T
    codes = jnp.reshape(_dist_argmin(xf, embt), (-1,))
    idx2d = jnp.reshape(codes, (_NW * _NCH, _CHUNK))
    quantized = _make_sc_gather()(embedding, idx2d)
    return (quantized, xf, codes)


# speedup vs baseline: 1.1993x; 1.1467x over previous
import functools

import jax
import jax.numpy as jnp
from jax import lax
from jax.experimental import pallas as pl
from jax.experimental.pallas import tpu as pltpu
from jax.experimental.pallas import tpu_sc as plsc

DIM = 32
K = 8192
N = 8192
TOKEN_BLOCK = 256


_CHUNK_K = 2048


def _dist_argmin_body(x_ref, embt_ref, codes_ref):
    xb = x_ref[...]
    embt = embt_ref[...]
    a_sq = jnp.sum(xb * xb, axis=-1, keepdims=True)
    b_sq = jnp.sum(embt * embt, axis=0, keepdims=True)
    prod2 = lax.dot_general(
        -2.0 * xb, embt, (((1,), (0,)), ((), ())),
        preferred_element_type=jnp.float32,
    )
    iota_f = lax.broadcasted_iota(
        jnp.int32, (xb.shape[0], _CHUNK_K), 1).astype(jnp.float32)
    best_v = None
    best_i = None
    for c in range(K // _CHUNK_K):
        dslice = (a_sq + b_sq[:, c * _CHUNK_K:(c + 1) * _CHUNK_K]) + \
            prod2[:, c * _CHUNK_K:(c + 1) * _CHUNK_K]
        cmin = jnp.min(dslice, axis=1, keepdims=True)
        cidx = jnp.min(jnp.where(dslice == cmin, iota_f, float(_CHUNK_K)),
                       axis=1, keepdims=True).astype(jnp.int32) + c * _CHUNK_K
        if c == 0:
            best_v, best_i = cmin, cidx
        else:
            keep = (best_v < cmin) | ((best_v == cmin) & (best_i < cidx))
            best_v = jnp.where(keep, best_v, cmin)
            best_i = jnp.where(keep, best_i, cidx)
        best_v = best_v.astype(jnp.bfloat16).astype(jnp.float32)
    codes_ref[...] = best_i


def _dist_argmin(xf, embt):
    return pl.pallas_call(
        _dist_argmin_body,
        grid=(N // TOKEN_BLOCK,),
        in_specs=[
            pl.BlockSpec((TOKEN_BLOCK, DIM), lambda i: (i, 0)),
            pl.BlockSpec((DIM, K), lambda i: (0, 0)),
        ],
        out_specs=pl.BlockSpec((TOKEN_BLOCK, 1), lambda i: (i, 0)),
        out_shape=jax.ShapeDtypeStruct((N, 1), jnp.int32),
    )(xf, embt)


_NC, _NS = 2, 16
_NW = _NC * _NS
_BPW = N // _NW
_CHUNK = 128
_NCH = _BPW // _CHUNK

@functools.lru_cache(maxsize=1)
def _make_sc_gather():
    mesh = plsc.VectorSubcoreMesh(core_axis_name="c", subcore_axis_name="s")

    @functools.partial(
        pl.kernel,
        mesh=mesh,
        out_type=jax.ShapeDtypeStruct((N, DIM), jnp.float32),
        scratch_types=[
            pltpu.VMEM((_NCH, _CHUNK), jnp.int32),
            pltpu.VMEM((_NCH, _CHUNK, DIM), jnp.float32),
            pltpu.SemaphoreType.DMA,
        ],
        compiler_params=pltpu.CompilerParams(use_tc_tiling_on_sc=False),
    )
    def _sc_gather(table_hbm, idx_hbm, out_hbm, idx_v, rows_v, sem):
        wid = lax.axis_index("s") * _NC + lax.axis_index("c")
        pltpu.sync_copy(idx_hbm.at[pl.ds(wid * _NCH, _NCH)], idx_v)
        copies = [
            pltpu.async_copy(table_hbm.at[idx_v.at[j]], rows_v.at[j], sem)
            for j in range(_NCH)
        ]
        for c in copies:
            c.wait()
        for j in range(_NCH):
            pltpu.sync_copy(
                rows_v.at[j], out_hbm.at[pl.ds(wid * _BPW + j * _CHUNK, _CHUNK)]
            )

    return _sc_gather


def kernel(x, embedding):
    xf = jnp.reshape(x, (-1, x.shape[-1]))
    embt = embedding.T
    codes = jnp.reshape(_dist_argmin(xf, embt), (-1,))
    idx2d = jnp.reshape(codes, (_NW * _NCH, _CHUNK))
    quantized = _make_sc_gather()(embedding, idx2d)
    return (quantized, xf, codes)


# speedup vs baseline: 1.2843x; 1.0709x over previous
import functools

import jax
import jax.numpy as jnp
from jax import lax
from jax.experimental import pallas as pl
from jax.experimental.pallas import tpu as pltpu
from jax.experimental.pallas import tpu_sc as plsc

DIM = 32
K = 8192
N = 8192
TOKEN_BLOCK = 1024


_CHUNK_K = 2048


def _dist_argmin_body(x_ref, embt_ref, codes_ref):
    xb = x_ref[...]
    embt = embt_ref[...]
    a_sq = jnp.sum(xb * xb, axis=-1, keepdims=True)
    b_sq = jnp.sum(embt * embt, axis=0, keepdims=True)
    prod2 = lax.dot_general(
        -2.0 * xb, embt, (((1,), (0,)), ((), ())),
        preferred_element_type=jnp.float32,
    )
    iota_f = lax.broadcasted_iota(
        jnp.int32, (xb.shape[0], _CHUNK_K), 1).astype(jnp.float32)
    best_v = None
    best_i = None
    for c in range(K // _CHUNK_K):
        dslice = (a_sq + b_sq[:, c * _CHUNK_K:(c + 1) * _CHUNK_K]) + \
            prod2[:, c * _CHUNK_K:(c + 1) * _CHUNK_K]
        cmin = jnp.min(dslice, axis=1, keepdims=True)
        cidx = jnp.min(jnp.where(dslice == cmin, iota_f, float(_CHUNK_K)),
                       axis=1, keepdims=True).astype(jnp.int32) + c * _CHUNK_K
        if c == 0:
            best_v, best_i = cmin, cidx
        else:
            keep = (best_v < cmin) | ((best_v == cmin) & (best_i < cidx))
            best_v = jnp.where(keep, best_v, cmin)
            best_i = jnp.where(keep, best_i, cidx)
        best_v = best_v.astype(jnp.bfloat16).astype(jnp.float32)
    codes_ref[...] = best_i


def _dist_argmin(xf, embt):
    return pl.pallas_call(
        _dist_argmin_body,
        grid=(N // TOKEN_BLOCK,),
        in_specs=[
            pl.BlockSpec((TOKEN_BLOCK, DIM), lambda i: (i, 0)),
            pl.BlockSpec((DIM, K), lambda i: (0, 0)),
        ],
        out_specs=pl.BlockSpec((TOKEN_BLOCK, 1), lambda i: (i, 0)),
        out_shape=jax.ShapeDtypeStruct((N, 1), jnp.int32),
    )(xf, embt)


_NC, _NS = 2, 16
_NW = _NC * _NS
_BPW = N // _NW
_CHUNK = 128
_NCH = _BPW // _CHUNK

@functools.lru_cache(maxsize=1)
def _make_sc_gather():
    mesh = plsc.VectorSubcoreMesh(core_axis_name="c", subcore_axis_name="s")

    @functools.partial(
        pl.kernel,
        mesh=mesh,
        out_type=jax.ShapeDtypeStruct((N, DIM), jnp.float32),
        scratch_types=[
            pltpu.VMEM((_NCH, _CHUNK), jnp.int32),
            pltpu.VMEM((_NCH, _CHUNK, DIM), jnp.float32),
            pltpu.SemaphoreType.DMA,
        ],
        compiler_params=pltpu.CompilerParams(use_tc_tiling_on_sc=False),
    )
    def _sc_gather(table_hbm, idx_hbm, out_hbm, idx_v, rows_v, sem):
        wid = lax.axis_index("s") * _NC + lax.axis_index("c")
        pltpu.sync_copy(idx_hbm.at[pl.ds(wid * _NCH, _NCH)], idx_v)
        copies = [
            pltpu.async_copy(table_hbm.at[idx_v.at[j]], rows_v.at[j], sem)
            for j in range(_NCH)
        ]
        for c in copies:
            c.wait()
        for j in range(_NCH):
            pltpu.sync_copy(
                rows_v.at[j], out_hbm.at[pl.ds(wid * _BPW + j * _CHUNK, _CHUNK)]
            )

    return _sc_gather


def kernel(x, embedding):
    xf = jnp.reshape(x, (-1, x.shape[-1]))
    embt = embedding.T
    codes = jnp.reshape(_dist_argmin(xf, embt), (-1,))
    idx2d = jnp.reshape(codes, (_NW * _NCH, _CHUNK))
    quantized = _make_sc_gather()(embedding, idx2d)
    return (quantized, xf, codes)
